# baseline (speedup 1.0000x reference)
import functools

import jax
import jax.numpy as jnp
from jax import lax
from jax.experimental import pallas as pl
from jax.experimental.pallas import tpu as pltpu
from jax.experimental.pallas import tpu_sc as plsc

_DIM = 64
_NC = 2
_NS = 16
_NW = _NC * _NS

_SEG = 128
_L = 16


_BLK = 256


def _detile_kernel(n_rows_each: int, dim: int):
    n_blocks = n_rows_each // _BLK
    tail = n_rows_each - n_blocks * _BLK
    assert tail % _L == 0
    n_units = 2 * n_blocks
    per_tile = (n_units + _NW - 1) // _NW

    mesh = plsc.VectorSubcoreMesh(core_axis_name="c", subcore_axis_name="s")

    @functools.partial(
        pl.kernel,
        out_type=jax.ShapeDtypeStruct((2 * n_rows_each * dim,), jnp.float32),
        mesh=mesh,
        scratch_types=[
            pltpu.VMEM((dim, _BLK), jnp.float32),
            pltpu.VMEM((dim, _BLK), jnp.float32),
            pltpu.VMEM((_BLK * dim,), jnp.float32),
            pltpu.VMEM((_BLK * dim,), jnp.float32),
            pltpu.VMEM((dim, 32), jnp.float32),
            pltpu.SemaphoreType.DMA,
            pltpu.SemaphoreType.DMA,
        ],
        compiler_params=pltpu.CompilerParams(
            use_tc_tiling_on_sc=True, needs_layout_passes=False),
    )
    def k(ft_hbm, vt_hbm, out_hbm, blk0, blk1, stage0, stage1, blkt,
          sem_r, sem_w):
        wid = lax.axis_index("s") * _NC + lax.axis_index("c")
        iota64 = lax.iota(jnp.int32, _L) * dim

        def load(u, bk):
            t = (u >= n_blocks).astype(jnp.int32)
            j = u - t * n_blocks
            c0 = j * _BLK

            @pl.when(t == 0)
            def _():
                pltpu.async_copy(ft_hbm.at[:, pl.ds(c0, _BLK)], bk, sem_r)

            @pl.when(t == 1)
            def _():
                pltpu.async_copy(vt_hbm.at[:, pl.ds(c0, _BLK)], bk, sem_r)

        def wait_load(bk):
            pltpu.make_async_copy(
                ft_hbm.at[:, pl.ds(0, _BLK)], bk, sem_r).wait()

        iota = lax.iota(jnp.int32, _L)
        colrot = [(iota + su) & (_L - 1) for su in range(_L)]
        colrot64 = [cr * dim + iota for cr in colrot]

        def transpose_block(bk, st, ngroups):
            def lblk(lb, c):
                l0 = lb * _L
                for db in range(dim // _L):
                    d0 = db * _L
                    drow_idx = d0 + iota
                    sbase = l0 * dim + d0
                    for su in range(_L):
                        vals = plsc.load_gather(
                            bk, [drow_idx, l0 + colrot[su]])
                        plsc.store_scatter(
                            st, [sbase + colrot64[su]], vals)
                return c

            lax.fori_loop(0, ngroups, lblk, 0)

        first = wid * per_tile
        limit = jnp.minimum(first + per_tile, n_units)

        @pl.when(first < limit)
        def _():
            load(first, blk0)

        bufs = ((blk0, stage0, blk1), (blk1, stage1, blk0))

        def body(i, carry):
            for sl in range(2):
                bk, st, nxt = bufs[sl]
                u = first + 2 * i + sl

                @pl.when(u < limit)
                def _(u=u, bk=bk, st=st, nxt=nxt):
                    wait_load(bk)

                    @pl.when(u + 1 < limit)
                    def _():
                        load(u + 1, nxt)

                    @pl.when(u - 2 >= first)
                    def _():
                        wait_write()

                    transpose_block(bk, st, _BLK // _L)

                    t = (u >= n_blocks).astype(jnp.int32)
                    j = u - t * n_blocks
                    row0 = t * n_rows_each + j * _BLK
                    pltpu.async_copy(
                        st, out_hbm.at[pl.ds(row0 * dim, _BLK * dim)],
                        sem_w)
            return carry

        def wait_write():
            pltpu.make_async_copy(
                stage0, out_hbm.at[pl.ds(0, _BLK * dim)], sem_w).wait()

        lax.fori_loop(0, (per_tile + 1) // 2, body, 0)

        n_mine = limit - first

        @pl.when(n_mine >= 2)
        def _():
            wait_write()

        @pl.when(n_mine >= 1)
        def _():
            wait_write()

        if tail:
            def transpose_tail():
                def lblk(lb, c):
                    l0 = lb * _L

                    def dblk(db, c2):
                        d0 = db * _L
                        drow_idx = d0 + iota
                        sbase = l0 * dim + d0
                        for su in range(_L):
                            vals = plsc.load_gather(
                                blkt, [drow_idx, l0 + colrot[su]])
                            plsc.store_scatter(
                                stage0, [sbase + colrot64[su]], vals)
                        return c2

                    return lax.fori_loop(0, dim // _L, dblk, c)

                lax.fori_loop(0, tail // _L, lblk, 0)

            def do_tail(src_hbm, t):
                c0 = n_blocks * _BLK
                pltpu.async_copy(
                    src_hbm.at[:, pl.ds(c0, tail)], blkt, sem_r).wait()
                transpose_tail()
                row0 = t * n_rows_each + c0
                pltpu.async_copy(
                    stage0.at[pl.ds(0, tail * dim)],
                    out_hbm.at[pl.ds(row0 * dim, tail * dim)],
                    sem_w).wait()

            @pl.when(wid == _NW - 2)
            def _():
                do_tail(ft_hbm, 0)

            @pl.when(wid == _NW - 1)
            def _():
                do_tail(vt_hbm, 1)

    return k


def _gather_kernel(n_idx: int, vocab: int, dim: int):
    k_per_w = n_idx // _NW
    assert k_per_w * _NW == n_idx and k_per_w % _SEG == 0
    n_groups = k_per_w // _L

    mesh = plsc.VectorSubcoreMesh(core_axis_name="c", subcore_axis_name="s")

    @functools.partial(
        pl.kernel,
        out_type=jax.ShapeDtypeStruct((2 * n_idx, dim), jnp.float32),
        mesh=mesh,
        scratch_types=[
            pltpu.VMEM((k_per_w,), jnp.int32),
            pltpu.VMEM((k_per_w,), jnp.int32),
            pltpu.VMEM((2, _SEG, dim), jnp.float32),
            pltpu.SemaphoreType.DMA,
            pltpu.SemaphoreType.DMA,
            pltpu.SemaphoreType.DMA,
        ],
        compiler_params=pltpu.CompilerParams(
            use_tc_tiling_on_sc=False, needs_layout_passes=False),
    )
    def k(table_hbm, idx_hbm, out_hbm,
          idx_v, pos_v, rows, sem_i, sem_g, sem_s):
        wid = lax.axis_index("s") * _NC + lax.axis_index("c")
        base = wid * k_per_w
        pltpu.async_copy(idx_hbm.at[pl.ds(base, k_per_w)], idx_v, sem_i).wait()

        iota = lax.iota(jnp.int32, _L)

        def pos_body(g, c):
            pos_v[pl.ds(g * _L, _L)] = 2 * ((base + g * _L) + iota)
            return c

        lax.fori_loop(0, n_groups, pos_body, 0)

        n_seg = k_per_w // _SEG

        def gather(s):
            return pltpu.make_async_copy(
                table_hbm.at[idx_v.at[pl.ds(s * _SEG, _SEG)]],
                rows.at[s % 2], sem_g)

        gather(jnp.int32(0)).start()

        def body(s, carry):
            gather(s).wait()

            @pl.when(s + 1 < n_seg)
            def _():
                gather(s + 1).start()

            pltpu.async_copy(
                rows.at[s % 2],
                out_hbm.at[pos_v.at[pl.ds(s * _SEG, _SEG)]], sem_s).wait()
            return carry

        lax.fori_loop(0, n_seg, body, 0)

    return k


def kernel(inputs, fixed_w, var_w):
    b, h = inputs.shape
    n_idx = b * h
    n_fixed = fixed_w.shape[0]
    vocab = n_fixed + var_w.shape[0]
    idx1d = inputs.reshape(n_idx).astype(jnp.int32)
    flat = _detile_kernel(n_fixed, _DIM)(fixed_w.T, var_w.T)
    table = flat.reshape(vocab, _DIM)
    out3 = _gather_kernel(n_idx, vocab, _DIM)(table, idx1d)
    return out3.reshape(n_idx, 2 * _DIM)[:, :_DIM].reshape(b, h, _DIM)


# speedup vs baseline: 1.0148x; 1.0148x over previous
import functools

import jax
import jax.numpy as jnp
from jax import lax
from jax.experimental import pallas as pl
from jax.experimental.pallas import tpu as pltpu
from jax.experimental.pallas import tpu_sc as plsc

_DIM = 64
_NC = 2
_NS = 16
_NW = _NC * _NS

_SEG = 128
_L = 16


_BLK = 256


def _detile_kernel(n_rows_each: int, dim: int):
    n_blocks = n_rows_each // _BLK
    tail = n_rows_each - n_blocks * _BLK
    assert tail % _L == 0
    n_units = 2 * n_blocks
    per_tile = (n_units + _NW - 1) // _NW

    mesh = plsc.VectorSubcoreMesh(core_axis_name="c", subcore_axis_name="s")

    @functools.partial(
        pl.kernel,
        out_type=jax.ShapeDtypeStruct((2 * n_rows_each * dim,), jnp.float32),
        mesh=mesh,
        scratch_types=[
            pltpu.VMEM((dim, _BLK), jnp.float32),
            pltpu.VMEM((dim, _BLK), jnp.float32),
            pltpu.VMEM((_BLK * dim,), jnp.float32),
            pltpu.VMEM((_BLK * dim,), jnp.float32),
            pltpu.VMEM((dim, 32), jnp.float32),
            pltpu.SemaphoreType.DMA,
            pltpu.SemaphoreType.DMA,
        ],
        compiler_params=pltpu.CompilerParams(
            use_tc_tiling_on_sc=True, needs_layout_passes=False),
    )
    def k(ft_hbm, vt_hbm, out_hbm, blk0, blk1, stage0, stage1, blkt,
          sem_r, sem_w):
        wid = lax.axis_index("s") * _NC + lax.axis_index("c")
        iota64 = lax.iota(jnp.int32, _L) * dim

        def load(u, bk):
            t = (u >= n_blocks).astype(jnp.int32)
            j = u - t * n_blocks
            c0 = j * _BLK

            @pl.when(t == 0)
            def _():
                pltpu.async_copy(ft_hbm.at[:, pl.ds(c0, _BLK)], bk, sem_r)

            @pl.when(t == 1)
            def _():
                pltpu.async_copy(vt_hbm.at[:, pl.ds(c0, _BLK)], bk, sem_r)

        def wait_load(bk):
            pltpu.make_async_copy(
                ft_hbm.at[:, pl.ds(0, _BLK)], bk, sem_r).wait()

        iota = lax.iota(jnp.int32, _L)
        colrot = [(iota + su) & (_L - 1) for su in range(_L)]
        colrot64 = [cr * dim + iota for cr in colrot]

        def transpose_block(bk, st, ngroups):
            def lblk(lb, c):
                l0 = lb * _L

                def dblk(db, c2):
                    d0 = db * _L
                    drow_idx = d0 + iota
                    sbase = l0 * dim + d0
                    for su in range(_L):
                        vals = plsc.load_gather(
                            bk, [drow_idx, l0 + colrot[su]])
                        plsc.store_scatter(
                            st, [sbase + colrot64[su]], vals)
                    return c2

                return lax.fori_loop(0, dim // _L, dblk, c)

            lax.fori_loop(0, ngroups, lblk, 0)

        first = wid * per_tile
        limit = jnp.minimum(first + per_tile, n_units)

        @pl.when(first < limit)
        def _():
            load(first, blk0)

        bufs = ((blk0, stage0, blk1), (blk1, stage1, blk0))

        def body(i, carry):
            for sl in range(2):
                bk, st, nxt = bufs[sl]
                u = first + 2 * i + sl

                @pl.when(u < limit)
                def _(u=u, bk=bk, st=st, nxt=nxt):
                    wait_load(bk)

                    @pl.when(u + 1 < limit)
                    def _():
                        load(u + 1, nxt)

                    @pl.when(u - 2 >= first)
                    def _():
                        wait_write()

                    transpose_block(bk, st, _BLK // _L)

                    t = (u >= n_blocks).astype(jnp.int32)
                    j = u - t * n_blocks
                    row0 = t * n_rows_each + j * _BLK
                    pltpu.async_copy(
                        st, out_hbm.at[pl.ds(row0 * dim, _BLK * dim)],
                        sem_w)
            return carry

        def wait_write():
            pltpu.make_async_copy(
                stage0, out_hbm.at[pl.ds(0, _BLK * dim)], sem_w).wait()

        lax.fori_loop(0, (per_tile + 1) // 2, body, 0)

        n_mine = limit - first

        @pl.when(n_mine >= 2)
        def _():
            wait_write()

        @pl.when(n_mine >= 1)
        def _():
            wait_write()

        if tail:
            def transpose_tail():
                def lblk(lb, c):
                    l0 = lb * _L

                    def dblk(db, c2):
                        d0 = db * _L
                        drow_idx = d0 + iota
                        sbase = l0 * dim + d0
                        for su in range(_L):
                            vals = plsc.load_gather(
                                blkt, [drow_idx, l0 + colrot[su]])
                            plsc.store_scatter(
                                stage0, [sbase + colrot64[su]], vals)
                        return c2

                    return lax.fori_loop(0, dim // _L, dblk, c)

                lax.fori_loop(0, tail // _L, lblk, 0)

            def do_tail(src_hbm, t):
                c0 = n_blocks * _BLK
                pltpu.async_copy(
                    src_hbm.at[:, pl.ds(c0, tail)], blkt, sem_r).wait()
                transpose_tail()
                row0 = t * n_rows_each + c0
                pltpu.async_copy(
                    stage0.at[pl.ds(0, tail * dim)],
                    out_hbm.at[pl.ds(row0 * dim, tail * dim)],
                    sem_w).wait()

            @pl.when(wid == _NW - 2)
            def _():
                do_tail(ft_hbm, 0)

            @pl.when(wid == _NW - 1)
            def _():
                do_tail(vt_hbm, 1)

    return k


def _gather_kernel(n_idx: int, vocab: int, dim: int):
    k_per_w = n_idx // _NW
    assert k_per_w * _NW == n_idx and k_per_w % _SEG == 0
    n_groups = k_per_w // _L

    mesh = plsc.VectorSubcoreMesh(core_axis_name="c", subcore_axis_name="s")

    @functools.partial(
        pl.kernel,
        out_type=jax.ShapeDtypeStruct((2 * n_idx, dim), jnp.float32),
        mesh=mesh,
        scratch_types=[
            pltpu.VMEM((k_per_w,), jnp.int32),
            pltpu.VMEM((k_per_w,), jnp.int32),
            pltpu.VMEM((2, _SEG, dim), jnp.float32),
            pltpu.SemaphoreType.DMA,
            pltpu.SemaphoreType.DMA,
            pltpu.SemaphoreType.DMA,
        ],
        compiler_params=pltpu.CompilerParams(
            use_tc_tiling_on_sc=False, needs_layout_passes=False),
    )
    def k(table_hbm, idx_hbm, out_hbm,
          idx_v, pos_v, rows, sem_i, sem_g, sem_s):
        wid = lax.axis_index("s") * _NC + lax.axis_index("c")
        base = wid * k_per_w
        pltpu.async_copy(idx_hbm.at[pl.ds(base, k_per_w)], idx_v, sem_i).wait()

        iota = lax.iota(jnp.int32, _L)

        def pos_body(g, c):
            pos_v[pl.ds(g * _L, _L)] = 2 * ((base + g * _L) + iota)
            return c

        lax.fori_loop(0, n_groups, pos_body, 0)

        n_seg = k_per_w // _SEG

        def gather(s):
            return pltpu.make_async_copy(
                table_hbm.at[idx_v.at[pl.ds(s * _SEG, _SEG)]],
                rows.at[s % 2], sem_g)

        gather(jnp.int32(0)).start()

        def body(s, carry):
            gather(s).wait()

            @pl.when(s + 1 < n_seg)
            def _():
                gather(s + 1).start()

            pltpu.async_copy(
                rows.at[s % 2],
                out_hbm.at[pos_v.at[pl.ds(s * _SEG, _SEG)]], sem_s).wait()
            return carry

        lax.fori_loop(0, n_seg, body, 0)

    return k


def kernel(inputs, fixed_w, var_w):
    b, h = inputs.shape
    n_idx = b * h
    n_fixed = fixed_w.shape[0]
    vocab = n_fixed + var_w.shape[0]
    idx1d = inputs.reshape(n_idx).astype(jnp.int32)
    flat = _detile_kernel(n_fixed, _DIM)(fixed_w.T, var_w.T)
    table = flat.reshape(vocab, _DIM)
    out3 = _gather_kernel(n_idx, vocab, _DIM)(table, idx1d)
    return out3.reshape(n_idx, 2 * _DIM)[:, :_DIM].reshape(b, h, _DIM)
